# Initial kernel scaffold; baseline (speedup 1.0000x reference)
import jax
import jax.numpy as jnp
from jax.experimental import pallas as pl


def kernel(x):
    raise NotImplementedError("write your pallas kernel here")


# speedup vs baseline: 2.1650x; 2.1650x over previous
import functools

import jax
import jax.numpy as jnp
from jax.experimental import pallas as pl
from jax.experimental.pallas import tpu as pltpu

OUT_GROUPS = 1024
GROUPS_PER_BLK = 128


def _group_sums_body(n_cols, cb, x_ref, s_ref, gs_ref, gsq_ref, np_ref):
    j = pl.program_id(1)
    xb = x_ref[...]
    col0 = j * cb
    cols = jax.lax.broadcasted_iota(jnp.int32, xb.shape, 1) + col0
    xb = jnp.where(cols < n_cols, xb, 0.0)
    s = s_ref[...]
    xsq = xb * xb
    xb16 = xb.astype(jnp.bfloat16)
    xsq16 = xsq.astype(jnp.bfloat16)
    dn = (((1,), (0,)), ((), ()))
    gs_ref[...] = jax.lax.dot_general(xb16, s, dn,
                                      preferred_element_type=jnp.float32)
    gsq_ref[...] = jax.lax.dot_general(xsq16, s, dn,
                                       preferred_element_type=jnp.float32)
    part = jnp.broadcast_to(jnp.sum(xsq, axis=1, keepdims=True),
                            np_ref.shape)

    @pl.when(j == 0)
    def _():
        np_ref[...] = part

    @pl.when(j != 0)
    def _():
        np_ref[...] += part


def _finalize_body(gs_ref, gsq_ref, np_ref, out_ref):
    gs = gs_ref[...]
    gsq = gsq_ref[...]
    norm = np_ref[:, :1]
    total = jnp.sum(gs, axis=1, keepdims=True)
    is_amp = jnp.all(jnp.abs(norm - 1.0) <= (1e-6 + 1e-5))
    out_ref[...] = jnp.where(is_amp, gsq, gs / total)


@jax.jit
def kernel(x):
    rows, n_cols = x.shape
    w = -(-n_cols // OUT_GROUPS)
    cb = w * GROUPS_PER_BLK
    nj = -(-OUT_GROUPS // GROUPS_PER_BLK)
    rb = min(256, rows)

    a = jax.lax.broadcasted_iota(jnp.int32, (cb, GROUPS_PER_BLK), 0)
    g = jax.lax.broadcasted_iota(jnp.int32, (cb, GROUPS_PER_BLK), 1)
    sel = ((a >= g * w) & (a < (g + 1) * w)).astype(jnp.bfloat16)

    gs, gsq, nparts = pl.pallas_call(
        functools.partial(_group_sums_body, n_cols, cb),
        grid=(rows // rb, nj),
        in_specs=[
            pl.BlockSpec((rb, cb), lambda i, j: (i, j)),
            pl.BlockSpec((cb, GROUPS_PER_BLK), lambda i, j: (0, 0)),
        ],
        out_specs=[
            pl.BlockSpec((rb, GROUPS_PER_BLK), lambda i, j: (i, j)),
            pl.BlockSpec((rb, GROUPS_PER_BLK), lambda i, j: (i, j)),
            pl.BlockSpec((rb, 128), lambda i, j: (i, 0)),
        ],
        out_shape=[
            jax.ShapeDtypeStruct((rows, OUT_GROUPS), jnp.float32),
            jax.ShapeDtypeStruct((rows, OUT_GROUPS), jnp.float32),
            jax.ShapeDtypeStruct((rows, 128), jnp.float32),
        ],
    )(x, sel)

    out = pl.pallas_call(
        _finalize_body,
        out_shape=jax.ShapeDtypeStruct((rows, OUT_GROUPS), jnp.float32),
    )(gs, gsq, nparts)
    return out
